# baseline (speedup 1.0000x reference)
import functools

import jax
import jax.numpy as jnp
from jax.experimental import pallas as pl
from jax.experimental.pallas import tpu as pltpu
from jax.experimental.pallas import tpu_sc as plsc

_LAMBDA_L2 = 1.0
_BINS = 256
_NC = 2
_NS = 16
_LANES = 16
_NW = _NC * _NS

_N = 128 * 1 * 192 * 84
_COLS = 84
_ROWS = _N // _COLS
_RPW = _ROWS // _NW
_RCHUNK = 96
_NCHUNK = _RPW // _RCHUNK
_CHUNK = _RCHUNK * _COLS
_VECS = _CHUNK // _LANES

_DENOM = float(_N * 128)


def _sc_hist_body(src_hbm, tar_hbm, out_hbm, sbuf, tbuf, cnt, ssq,
                  cnt_f, ssq_f, sem0, sem1):
    wid = jax.lax.axis_index("s") * _NC + jax.lax.axis_index("c")
    base = wid * _RPW
    sems = (sem0, sem1)

    def start(c, slot):
        r = base + c * _RCHUNK
        img, r0 = r // 192, r % 192
        pltpu.async_copy(
            src_hbm.at[img, 0, pl.ds(r0, _RCHUNK)], sbuf.at[slot], sems[slot])
        pltpu.async_copy(
            tar_hbm.at[pl.ds(r, _RCHUNK)], tbuf.at[slot], sems[slot])

    def drain(c, slot):
        r = base + c * _RCHUNK
        img, r0 = r // 192, r % 192
        pltpu.make_async_copy(
            src_hbm.at[img, 0, pl.ds(r0, _RCHUNK)], sbuf.at[slot], sems[slot]).wait()
        pltpu.make_async_copy(
            tar_hbm.at[pl.ds(r, _RCHUNK)], tbuf.at[slot], sems[slot]).wait()

    zeros = jnp.zeros((_LANES,), jnp.float32)

    start(0, 0)

    @plsc.parallel_loop(0, _BINS, unroll=8)
    def _(i):
        cnt[pl.ds(i * _LANES, _LANES)] = zeros
        ssq[pl.ds(i * _LANES, _LANES)] = zeros

    lane = jax.lax.broadcasted_iota(jnp.int32, (_LANES,), 0)
    ones = jnp.ones((_LANES,), jnp.float32)
    tail_mask = lane >= 12

    for c in range(_NCHUNK):
        slot = c % 2
        if c + 1 < _NCHUNK:
            start(c + 1, 1 - slot)
        drain(c, slot)

        @plsc.parallel_loop(0, _RCHUNK, unroll=4)
        def _(r, slot=slot):
            for v in range(6):
                col = 16 * v if v < 5 else _COLS - _LANES
                t = tbuf[slot, r, pl.ds(col, _LANES)]
                s = sbuf[slot, r, pl.ds(col, _LANES)]
                idx = t.astype(jnp.int32) * _LANES + lane
                d = s - t
                if v < 5:
                    plsc.addupdate_scatter(cnt, [idx], ones)
                    plsc.addupdate_scatter(ssq, [idx], d * d)
                else:
                    plsc.addupdate_scatter(cnt, [idx], ones, mask=tail_mask)
                    plsc.addupdate_scatter(ssq, [idx], d * d, mask=tail_mask)

    @plsc.parallel_loop(0, _BINS // _LANES, unroll=2)
    def _(g):
        b0 = g * _LANES * _LANES
        accc = zeros
        accs = zeros
        for l in range(_LANES):
            idx = b0 + lane * _LANES + l
            accc = accc + plsc.load_gather(cnt, [idx])
            accs = accs + plsc.load_gather(ssq, [idx])
        cnt_f[pl.ds(g * _LANES, _LANES)] = accc
        ssq_f[pl.ds(g * _LANES, _LANES)] = accs

    pltpu.sync_copy(cnt_f, out_hbm.at[wid, 0])
    pltpu.sync_copy(ssq_f, out_hbm.at[wid, 1])


@jax.jit
def _sc_hist(src_rows, tar_rows):
    mesh = plsc.VectorSubcoreMesh(core_axis_name="c", subcore_axis_name="s")
    return pl.kernel(
        _sc_hist_body,
        out_type=jax.ShapeDtypeStruct((_NW, 2, _BINS), jnp.float32),
        mesh=mesh,
        compiler_params=pltpu.CompilerParams(
            needs_layout_passes=False, use_tc_tiling_on_sc=True),
        scratch_types=[
            pltpu.VMEM((2, _RCHUNK, _COLS), jnp.float32),
            pltpu.VMEM((2, _RCHUNK, _COLS), jnp.float32),
            pltpu.VMEM((_BINS * _LANES,), jnp.float32),
            pltpu.VMEM((_BINS * _LANES,), jnp.float32),
            pltpu.VMEM((_BINS,), jnp.float32),
            pltpu.VMEM((_BINS,), jnp.float32),
            pltpu.SemaphoreType.DMA,
            pltpu.SemaphoreType.DMA,
        ],
    )(src_rows, tar_rows)


def _epilogue_body(p_ref, o_ref):
    x = p_ref[...]
    cnt = jnp.sum(x[:, 0, :], axis=0)
    ssq = jnp.sum(x[:, 1, :], axis=0)
    p_y = cnt / _DENOM
    w = 1.0 / (p_y + 1e-12)
    loss = _LAMBDA_L2 * jnp.sum(w * ssq) / jnp.sum(w)
    o_ref[0, 0] = loss


def _epilogue(partials, interpret=False):
    return pl.pallas_call(
        _epilogue_body,
        out_shape=jax.ShapeDtypeStruct((1, 1), jnp.float32),
        out_specs=pl.BlockSpec(memory_space=pltpu.MemorySpace.SMEM),
        interpret=interpret,
    )(partials)


def kernel(src, tar):
    partials = _sc_hist(src, tar.reshape(_ROWS, _COLS))
    loss = _epilogue(partials)
    return loss[0, 0]


# speedup vs baseline: 1.0452x; 1.0452x over previous
import functools

import jax
import jax.numpy as jnp
from jax.experimental import pallas as pl
from jax.experimental.pallas import tpu as pltpu
from jax.experimental.pallas import tpu_sc as plsc

_LAMBDA_L2 = 1.0
_BINS = 256
_NC = 2
_NS = 16
_LANES = 16
_NW = _NC * _NS

_N = 128 * 1 * 192 * 84
_COLS = 84
_ROWS = _N // _COLS
_RPW = _ROWS // _NW
_RCHUNK = 96
_NCHUNK = _RPW // _RCHUNK
_CHUNK = _RCHUNK * _COLS
_VECS = _CHUNK // _LANES

_DENOM = float(_N * 128)


def _sc_hist_body(src_hbm, tar_hbm, out_hbm, sbuf, tbuf, cnt, ssq,
                  cnt_f, ssq_f, sem0, sem1):
    wid = jax.lax.axis_index("s") * _NC + jax.lax.axis_index("c")
    base = wid * _RPW
    sems = (sem0, sem1)

    def start(c, slot):
        r = base + c * _RCHUNK
        img, r0 = r // 192, r % 192
        pltpu.async_copy(
            src_hbm.at[img, 0, pl.ds(r0, _RCHUNK)], sbuf.at[slot], sems[slot])
        pltpu.async_copy(
            tar_hbm.at[pl.ds(r, _RCHUNK)], tbuf.at[slot], sems[slot])

    def drain(c, slot):
        r = base + c * _RCHUNK
        img, r0 = r // 192, r % 192
        pltpu.make_async_copy(
            src_hbm.at[img, 0, pl.ds(r0, _RCHUNK)], sbuf.at[slot], sems[slot]).wait()
        pltpu.make_async_copy(
            tar_hbm.at[pl.ds(r, _RCHUNK)], tbuf.at[slot], sems[slot]).wait()

    zeros = jnp.zeros((_LANES,), jnp.float32)

    start(0, 0)

    @plsc.parallel_loop(0, _BINS, unroll=8)
    def _(i):
        cnt[pl.ds(i * _LANES, _LANES)] = zeros
        ssq[pl.ds(i * _LANES, _LANES)] = zeros

    lane = jax.lax.broadcasted_iota(jnp.int32, (_LANES,), 0)
    ones = jnp.ones((_LANES,), jnp.float32)
    tail_mask = lane >= 12

    for c in range(_NCHUNK):
        slot = c % 2
        if c + 1 < _NCHUNK:
            start(c + 1, 1 - slot)
        drain(c, slot)

        @plsc.parallel_loop(0, _RCHUNK, unroll=2)
        def _(r, slot=slot):
            for v in range(6):
                col = 16 * v if v < 5 else _COLS - _LANES
                t = tbuf[slot, r, pl.ds(col, _LANES)]
                s = sbuf[slot, r, pl.ds(col, _LANES)]
                idx = t.astype(jnp.int32) * _LANES + lane
                d = s - t
                if v < 5:
                    plsc.addupdate_scatter(cnt, [idx], ones)
                    plsc.addupdate_scatter(ssq, [idx], d * d)
                else:
                    plsc.addupdate_scatter(cnt, [idx], ones, mask=tail_mask)
                    plsc.addupdate_scatter(ssq, [idx], d * d, mask=tail_mask)

    @plsc.parallel_loop(0, _BINS // _LANES, unroll=2)
    def _(g):
        b0 = g * _LANES * _LANES
        accc = zeros
        accs = zeros
        for l in range(_LANES):
            idx = b0 + lane * _LANES + l
            accc = accc + plsc.load_gather(cnt, [idx])
            accs = accs + plsc.load_gather(ssq, [idx])
        cnt_f[pl.ds(g * _LANES, _LANES)] = accc
        ssq_f[pl.ds(g * _LANES, _LANES)] = accs

    pltpu.sync_copy(cnt_f, out_hbm.at[wid, 0])
    pltpu.sync_copy(ssq_f, out_hbm.at[wid, 1])


@jax.jit
def _sc_hist(src_rows, tar_rows):
    mesh = plsc.VectorSubcoreMesh(core_axis_name="c", subcore_axis_name="s")
    return pl.kernel(
        _sc_hist_body,
        out_type=jax.ShapeDtypeStruct((_NW, 2, _BINS), jnp.float32),
        mesh=mesh,
        compiler_params=pltpu.CompilerParams(
            needs_layout_passes=False, use_tc_tiling_on_sc=True),
        scratch_types=[
            pltpu.VMEM((2, _RCHUNK, _COLS), jnp.float32),
            pltpu.VMEM((2, _RCHUNK, _COLS), jnp.float32),
            pltpu.VMEM((_BINS * _LANES,), jnp.float32),
            pltpu.VMEM((_BINS * _LANES,), jnp.float32),
            pltpu.VMEM((_BINS,), jnp.float32),
            pltpu.VMEM((_BINS,), jnp.float32),
            pltpu.SemaphoreType.DMA,
            pltpu.SemaphoreType.DMA,
        ],
    )(src_rows, tar_rows)


def _epilogue_body(p_ref, o_ref):
    x = p_ref[...]
    cnt = jnp.sum(x[:, 0, :], axis=0)
    ssq = jnp.sum(x[:, 1, :], axis=0)
    p_y = cnt / _DENOM
    w = 1.0 / (p_y + 1e-12)
    loss = _LAMBDA_L2 * jnp.sum(w * ssq) / jnp.sum(w)
    o_ref[0, 0] = loss


def _epilogue(partials, interpret=False):
    return pl.pallas_call(
        _epilogue_body,
        out_shape=jax.ShapeDtypeStruct((1, 1), jnp.float32),
        out_specs=pl.BlockSpec(memory_space=pltpu.MemorySpace.SMEM),
        interpret=interpret,
    )(partials)


def kernel(src, tar):
    partials = _sc_hist(src, tar.reshape(_ROWS, _COLS))
    loss = _epilogue(partials)
    return loss[0, 0]
